# baseline (speedup 1.0000x reference)
import functools

import jax
import jax.numpy as jnp
from jax import lax
from jax.experimental import pallas as pl
from jax.experimental.pallas import tpu as pltpu
from jax.experimental.pallas import tpu_sc as plsc

VOCAB = 1000000
EMB = 64
PADW = 128
B = 4096
L = 50

NC = 2
NS = 16
NW = NC * NS

BPW = B // NW
R = 4
RO = 2
LANES = 16
JJ = EMB // LANES


def _emb_body(x_hbm, xp_hbm, emb_hbm, pos_hbm, out_hbm, *refs):
    idx_v, xp_v, pos_v = refs[0], refs[1], refs[2]
    gslots = refs[3 : 3 + R]
    oslots = refs[3 + R : 3 + R + RO]
    sems = refs[3 + R + RO :]
    gsems, osems = sems[:R], sems[R:]

    wid = lax.axis_index("s") * NC + lax.axis_index("c")
    b_base = wid * BPW

    pltpu.sync_copy(x_hbm.at[wid], idx_v)
    pltpu.sync_copy(xp_hbm.at[wid], xp_v)
    pltpu.sync_copy(pos_hbm, pos_v)
    pos_regs = [pos_v[pl.ds(j * LANES, LANES)] for j in range(JJ)]
    iota = lax.iota(jnp.int32, LANES)
    col_regs = [
        jnp.full((LANES,), j * LANES, jnp.int32) + iota for j in range(JJ)
    ]

    def fire(b, q):
        pltpu.async_copy(emb_hbm.at[xp_v.at[b]], gslots[q], gsems[q])

    def wait_gather(b, q):
        pltpu.make_async_copy(
            emb_hbm.at[xp_v.at[b]], gslots[q], gsems[q]
        ).wait()

    def wait_out(p, b):
        pltpu.make_async_copy(oslots[p], out_hbm.at[b], osems[p]).wait()

    for q in range(R):
        fire(q, q)

    def do_batch(b, q, p):
        @pl.when(b >= RO)
        def _():
            wait_out(p, b_base + b - RO)

        wait_gather(b, q)

        bfull = jnp.full((LANES,), b, jnp.int32)
        for r in range(L):
            rfull = jnp.full((LANES,), r, jnp.int32)
            xb = plsc.load_gather(idx_v, [bfull, rfull])
            sub = lax.shift_left(lax.bitwise_and(xb, 1), 6)
            for j in range(JJ):
                v = plsc.load_gather(gslots[q], [rfull, sub + col_regs[j]])
                oslots[p][r, pl.ds(j * LANES, LANES)] = v + pos_regs[j]

        @pl.when(b < BPW - R)
        def _():
            fire(b + R, q)

        pltpu.async_copy(oslots[p], out_hbm.at[b_base + b], osems[p])

    def outer(bb, carry):
        for q in range(R):
            do_batch(bb * R + q, q, q % RO)
        return carry

    lax.fori_loop(0, BPW // R, outer, 0)

    wait_out(0, b_base + BPW - 2)
    wait_out(1, b_base + BPW - 1)


@jax.jit
def _emb_lookup(x3, xp3, embc, w_pos):
    mesh = plsc.VectorSubcoreMesh(core_axis_name="c", subcore_axis_name="s")
    f = functools.partial(
        pl.kernel,
        mesh=mesh,
        out_type=jax.ShapeDtypeStruct((B, L, EMB), jnp.float32),
        scratch_types=[
            pltpu.VMEM((BPW, L), jnp.int32),
            pltpu.VMEM((BPW, L), jnp.int32),
            pltpu.VMEM((EMB,), jnp.float32),
        ]
        + [pltpu.VMEM((L, PADW), jnp.float32) for _ in range(R)]
        + [pltpu.VMEM((L, EMB), jnp.float32) for _ in range(RO)]
        + [pltpu.SemaphoreType.DMA] * (R + RO),
        compiler_params=pltpu.CompilerParams(
            use_tc_tiling_on_sc=True, needs_layout_passes=False
        ),
    )(_emb_body)
    return f(x3, xp3, embc, w_pos)


def kernel(x, W_emb, W_pos):
    x3 = jnp.reshape(x.astype(jnp.int32), (NW, BPW, L))
    xp3 = lax.shift_right_logical(x3, 1)
    embc = jnp.reshape(W_emb, (VOCAB // 2, PADW))
    return _emb_lookup(x3, xp3, embc, W_pos)


# speedup vs baseline: 1.3961x; 1.3961x over previous
import functools

import jax
import jax.numpy as jnp
from jax import lax
from jax.experimental import pallas as pl
from jax.experimental.pallas import tpu as pltpu
from jax.experimental.pallas import tpu_sc as plsc

VOCAB = 1000000
EMB = 64
PADW = 128
B = 4096
L = 50

NC = 2
NS = 16
NW = NC * NS

BPW = B // NW
R = 4
RO = 2
LANES = 16
JJ = EMB // LANES


def _emb_body(x_hbm, emb_hbm, pos_hbm, out_hbm, *refs):
    idx_v, pos_v = refs[0], refs[1]
    gslots = refs[2 : 2 + R]
    oslots = refs[2 + R : 2 + R + RO]
    sems = refs[2 + R + RO :]
    gsems, osems = sems[:R], sems[R:]

    wid = lax.axis_index("s") * NC + lax.axis_index("c")
    b_base = wid * BPW

    pltpu.sync_copy(x_hbm.at[wid], idx_v)
    pltpu.sync_copy(pos_hbm, pos_v)
    pos_regs = [pos_v[pl.ds(j * LANES, LANES)] for j in range(JJ)]

    def fire(b, q):
        pltpu.async_copy(emb_hbm.at[idx_v.at[b]], gslots[q], gsems[q])

    def wait_gather(b, q):
        pltpu.make_async_copy(
            emb_hbm.at[idx_v.at[b]], gslots[q], gsems[q]
        ).wait()

    def wait_out(p, b):
        pltpu.make_async_copy(oslots[p], out_hbm.at[b], osems[p]).wait()

    for q in range(R):
        fire(q, q)

    def do_batch(b, q, p):
        @pl.when(b >= RO)
        def _():
            wait_out(p, b_base + b - RO)

        wait_gather(b, q)

        for r in range(L):
            for j in range(JJ):
                v = gslots[q][r, pl.ds(j * LANES, LANES)]
                oslots[p][r, pl.ds(j * LANES, LANES)] = v + pos_regs[j]

        @pl.when(b < BPW - R)
        def _():
            fire(b + R, q)

        pltpu.async_copy(oslots[p], out_hbm.at[b_base + b], osems[p])

    def outer(bb, carry):
        for q in range(R):
            do_batch(bb * R + q, q, q % RO)
        return carry

    lax.fori_loop(0, BPW // R, outer, 0)

    wait_out(0, b_base + BPW - 2)
    wait_out(1, b_base + BPW - 1)


@jax.jit
def _emb_lookup(x3, embp, w_pos):
    mesh = plsc.VectorSubcoreMesh(core_axis_name="c", subcore_axis_name="s")
    f = functools.partial(
        pl.kernel,
        mesh=mesh,
        out_type=jax.ShapeDtypeStruct((B, L, EMB), jnp.float32),
        scratch_types=[
            pltpu.VMEM((BPW, L), jnp.int32),
            pltpu.VMEM((EMB,), jnp.float32),
        ]
        + [pltpu.VMEM((L, PADW), jnp.float32) for _ in range(R)]
        + [pltpu.VMEM((L, EMB), jnp.float32) for _ in range(RO)]
        + [pltpu.SemaphoreType.DMA] * (R + RO),
        compiler_params=pltpu.CompilerParams(
            use_tc_tiling_on_sc=True, needs_layout_passes=False
        ),
    )(_emb_body)
    return f(x3, embp, w_pos)


def kernel(x, W_emb, W_pos):
    x3 = jnp.reshape(x.astype(jnp.int32), (NW, BPW, L))
    embp = jnp.pad(W_emb, ((0, 0), (0, PADW - EMB)))
    return _emb_lookup(x3, embp, W_pos)
